# baseline (speedup 1.0000x reference)
import functools

import jax
import jax.numpy as jnp
from jax.experimental import pallas as pl
from jax.experimental.pallas import tpu as pltpu
from jax.experimental.pallas import tpu_sc as plsc

TEMP_INV = 1.0 / 0.07
B = 4096
D = 128
N = 100000
CHUNK = 128
N_PAD = 100352
G = N_PAD // CHUNK
CB = 1024
GC = CB // CHUNK
K = 10
RB = 512
HB = B // 2
NEG = -1.0e30
MASKV = -3.0e38
BIGI = 1 << 30


def _l2n(x):
    return x * (1.0 / jnp.maximum(jnp.sqrt(jnp.sum(x * x, axis=1, keepdims=True)), 1e-12))


def _p1_body(a_ref, p_ref, n_ref, sims_ref, cmax_ref, pd_ref, an_scr):
    j = pl.program_id(0)

    @pl.when(j == 0)
    def _():
        an = _l2n(a_ref[...])
        pd_ref[...] = an * _l2n(p_ref[...])
        an_scr[...] = an.astype(jnp.bfloat16)

    n = n_ref[...]
    nn = _l2n(n).astype(jnp.bfloat16)
    col = j * CB + jax.lax.broadcasted_iota(jnp.int32, (1, CB), 1)
    an = an_scr[...]
    for h in range(B // HB):
        s = jax.lax.dot_general(
            an[h * HB:(h + 1) * HB, :], nn, (((1,), (1,)), ((), ())),
            preferred_element_type=jnp.float32,
        ).astype(jnp.bfloat16)
        s = jnp.where(col < N, s, jnp.bfloat16(NEG))
        maxcols = []
        hb16 = HB // 16
        for c in range(GC):
            sc_vals = s[:, c * CHUNK:(c + 1) * CHUNK]
            w = pltpu.bitcast(sc_vals, jnp.int32)
            sims_ref[h * hb16:(h + 1) * hb16, c, :, :] = w.reshape(hb16, 8, CHUNK)
            maxcols.append(jnp.max(sc_vals, axis=1, keepdims=True).astype(jnp.float32))
        cmax_ref[:, h * HB:(h + 1) * HB] = jnp.transpose(
            jnp.concatenate(maxcols, axis=1))


def _phase1(anchor, positive, n_pad):
    grid = N_PAD // CB
    return pl.pallas_call(
        _p1_body,
        grid=(grid,),
        in_specs=[
            pl.BlockSpec((B, D), lambda j: (0, 0)),
            pl.BlockSpec((B, D), lambda j: (0, 0)),
            pl.BlockSpec((CB, D), lambda j: (j, 0)),
        ],
        out_specs=(
            pl.BlockSpec((B // 16, GC, 8, CHUNK), lambda j: (0, j, 0, 0)),
            pl.BlockSpec((GC, B), lambda j: (j, 0)),
            pl.BlockSpec((B, D), lambda j: (0, 0)),
        ),
        out_shape=(
            jax.ShapeDtypeStruct((B // 16, G, 8, CHUNK), jnp.int32),
            jax.ShapeDtypeStruct((G, B), jnp.float32),
            jax.ShapeDtypeStruct((B, D), jnp.float32),
        ),
        scratch_shapes=[pltpu.VMEM((B, D), jnp.bfloat16)],
        compiler_params=pltpu.CompilerParams(
            dimension_semantics=("arbitrary",),
        ),
    )(anchor, positive, n_pad)


def _p2_body(cmax_ref, idx_ref):
    i = pl.program_id(0)
    x = cmax_ref[...]
    chunkio = jax.lax.broadcasted_iota(jnp.int32, (G, RB), 0)
    rows = i * RB + jax.lax.broadcasted_iota(jnp.int32, (1, RB), 1)
    rowbase = (rows // 16) * (G * 8) + (rows % 16) // 2
    parts = []
    for _ in range(K):
        m = jnp.max(x, axis=0, keepdims=True)
        fi = jnp.min(jnp.where(x == m, chunkio, BIGI), axis=0, keepdims=True)
        parts.append(rowbase + fi * 8)
        x = jnp.where(chunkio == fi, MASKV, x)
    parts.append(jnp.zeros((16 - K, RB), jnp.int32))
    idx_ref[...] = jnp.concatenate(parts, axis=0)


def _phase2(cmax):
    grid = B // RB
    return pl.pallas_call(
        _p2_body,
        grid=(grid,),
        in_specs=[pl.BlockSpec((G, RB), lambda i: (0, i))],
        out_specs=pl.BlockSpec((16, RB), lambda i: (0, i)),
        out_shape=jax.ShapeDtypeStruct((16, B), jnp.int32),
        compiler_params=pltpu.CompilerParams(
            dimension_semantics=("arbitrary",),
        ),
    )(cmax)


def _sc_gather(table, flat_idx):
    n_idx = B * K
    info = plsc.get_sparse_core_info()
    nw = info.num_cores * info.num_subcores
    per_w = n_idx // nw
    n_grp = per_w // CHUNK
    mesh = plsc.VectorSubcoreMesh(core_axis_name="c", subcore_axis_name="s")

    @functools.partial(
        pl.kernel,
        mesh=mesh,
        out_type=jax.ShapeDtypeStruct((n_idx, CHUNK), jnp.int32),
        scratch_types=[
            pltpu.VMEM((CHUNK,), jnp.int32),
            pltpu.VMEM((CHUNK,), jnp.int32),
            pltpu.VMEM((CHUNK, CHUNK), jnp.int32),
            pltpu.VMEM((CHUNK, CHUNK), jnp.int32),
            pltpu.SemaphoreType.DMA,
            pltpu.SemaphoreType.DMA,
        ],
    )
    def gk(table_hbm, idx_hbm, out_hbm, idx0, idx1, rows0, rows1, sem0, sem1):
        wid = jax.lax.axis_index("s") * info.num_cores + jax.lax.axis_index("c")
        base = wid * per_w
        idxs, rows, sems = (idx0, idx1), (rows0, rows1), (sem0, sem1)
        pltpu.sync_copy(idx_hbm.at[pl.ds(base, CHUNK)], idx0)
        prev = pltpu.async_copy(table_hbm.at[idx0], rows0, sem0)
        for j in range(1, n_grp):
            b = j % 2
            off = base + j * CHUNK
            pltpu.sync_copy(idx_hbm.at[pl.ds(off, CHUNK)], idxs[b])
            cur = pltpu.async_copy(table_hbm.at[idxs[b]], rows[b], sems[b])
            prev.wait()
            pltpu.sync_copy(rows[1 - b], out_hbm.at[pl.ds(off - CHUNK, CHUNK)])
            prev = cur
        prev.wait()
        pltpu.sync_copy(rows[(n_grp - 1) % 2],
                        out_hbm.at[pl.ds(base + (n_grp - 1) * CHUNK, CHUNK)])

    return gk(table, flat_idx)


def _p4_body(cand_ref, pd_ref, out_ref):
    i = pl.program_id(0)
    cw = cand_ref[...]
    par = jax.lax.broadcasted_iota(jnp.int32, (RB, 1), 0) % 2
    bits = jnp.where(par == 0, cw << 16, cw & jnp.int32(-65536))
    x = jax.lax.bitcast_convert_type(bits, jnp.float32) * TEMP_INV
    pos = jnp.sum(pd_ref[...], axis=1, keepdims=True) * TEMP_INV
    colio = jax.lax.broadcasted_iota(jnp.int32, (RB, K * CHUNK), 1)
    vals = []
    for k in range(K):
        m = jnp.max(x, axis=1, keepdims=True)
        vals.append(m)
        if k < K - 1:
            fi = jnp.min(jnp.where(x == m, colio, BIGI), axis=1, keepdims=True)
            x = jnp.where(colio == fi, MASKV, x)
    mx = jnp.maximum(vals[0], pos)
    ssum = jnp.exp(pos - mx)
    for k in range(K):
        ssum = ssum + jnp.exp(vals[k] - mx)
    per_row = mx + jnp.log(ssum) - pos
    contrib = (jnp.sum(per_row) * (1.0 / B)).reshape(1, 1)

    @pl.when(i == 0)
    def _():
        out_ref[...] = jnp.zeros((1, 1), jnp.float32)

    out_ref[...] += contrib


def _phase4(cand, posdot):
    grid = B // RB
    return pl.pallas_call(
        _p4_body,
        grid=(grid,),
        in_specs=[
            pl.BlockSpec((RB, K * CHUNK), lambda i: (i, 0)),
            pl.BlockSpec((RB, D), lambda i: (i, 0)),
        ],
        out_specs=pl.BlockSpec((1, 1), lambda i: (0, 0)),
        out_shape=jax.ShapeDtypeStruct((1, 1), jnp.float32),
        compiler_params=pltpu.CompilerParams(
            dimension_semantics=("arbitrary",),
        ),
    )(cand, posdot)


def kernel(anchor, positive, negative_pool):
    n_pad = jnp.pad(negative_pool, ((0, N_PAD - N), (0, 0)))
    sims, cmax, posdot = _phase1(anchor, positive, n_pad)
    idx16 = _phase2(cmax)
    flat_idx = idx16[:K, :].T.reshape(-1)
    table = sims.reshape(-1, CHUNK)
    cand = _sc_gather(table, flat_idx)
    loss = _phase4(cand.reshape(B, K * CHUNK), posdot)
    return loss[0, 0]


# speedup vs baseline: 1.0288x; 1.0288x over previous
import functools

import jax
import jax.numpy as jnp
from jax.experimental import pallas as pl
from jax.experimental.pallas import tpu as pltpu
from jax.experimental.pallas import tpu_sc as plsc

TEMP_INV = 1.0 / 0.07
B = 4096
D = 128
N = 100000
CHUNK = 128
N_PAD = 100352
G = N_PAD // CHUNK
CB = 1024
GC = CB // CHUNK
K = 10
RB = 512
HB = B
NEG = -1.0e30
MASKV = -3.0e38
BIGI = 1 << 30


def _l2n(x):
    return x * (1.0 / jnp.maximum(jnp.sqrt(jnp.sum(x * x, axis=1, keepdims=True)), 1e-12))


def _p1_body(a_ref, p_ref, n_ref, sims_ref, cmax_ref, pd_ref, an_scr):
    j = pl.program_id(0)

    @pl.when(j == 0)
    def _():
        an = _l2n(a_ref[...])
        pd_ref[...] = an * _l2n(p_ref[...])
        an_scr[...] = an.astype(jnp.bfloat16)

    n = n_ref[...]
    nn = _l2n(n).astype(jnp.bfloat16)
    col = j * CB + jax.lax.broadcasted_iota(jnp.int32, (1, CB), 1)
    an = an_scr[...]
    for h in range(B // HB):
        s = jax.lax.dot_general(
            an[h * HB:(h + 1) * HB, :], nn, (((1,), (1,)), ((), ())),
            preferred_element_type=jnp.float32,
        ).astype(jnp.bfloat16)
        s = jnp.where(col < N, s, jnp.bfloat16(NEG))
        maxcols = []
        hb16 = HB // 16
        for c in range(GC):
            sc_vals = s[:, c * CHUNK:(c + 1) * CHUNK]
            w = pltpu.bitcast(sc_vals, jnp.int32)
            sims_ref[h * hb16:(h + 1) * hb16, c, :, :] = w.reshape(hb16, 8, CHUNK)
            maxcols.append(jnp.max(sc_vals, axis=1, keepdims=True).astype(jnp.float32))
        cmax_ref[:, h * HB:(h + 1) * HB] = jnp.transpose(
            jnp.concatenate(maxcols, axis=1))


def _phase1(anchor, positive, n_pad):
    grid = N_PAD // CB
    return pl.pallas_call(
        _p1_body,
        grid=(grid,),
        in_specs=[
            pl.BlockSpec((B, D), lambda j: (0, 0)),
            pl.BlockSpec((B, D), lambda j: (0, 0)),
            pl.BlockSpec((CB, D), lambda j: (j, 0)),
        ],
        out_specs=(
            pl.BlockSpec((B // 16, GC, 8, CHUNK), lambda j: (0, j, 0, 0)),
            pl.BlockSpec((GC, B), lambda j: (j, 0)),
            pl.BlockSpec((B, D), lambda j: (0, 0)),
        ),
        out_shape=(
            jax.ShapeDtypeStruct((B // 16, G, 8, CHUNK), jnp.int32),
            jax.ShapeDtypeStruct((G, B), jnp.float32),
            jax.ShapeDtypeStruct((B, D), jnp.float32),
        ),
        scratch_shapes=[pltpu.VMEM((B, D), jnp.bfloat16)],
        compiler_params=pltpu.CompilerParams(
            dimension_semantics=("arbitrary",),
        ),
    )(anchor, positive, n_pad)


def _p2_body(cmax_ref, idx_ref):
    i = pl.program_id(0)
    x = cmax_ref[...]
    chunkio = jax.lax.broadcasted_iota(jnp.int32, (G, RB), 0)
    rows = i * RB + jax.lax.broadcasted_iota(jnp.int32, (1, RB), 1)
    rowbase = (rows // 16) * (G * 8) + (rows % 16) // 2
    parts = []
    for _ in range(K):
        m = jnp.max(x, axis=0, keepdims=True)
        fi = jnp.min(jnp.where(x == m, chunkio, BIGI), axis=0, keepdims=True)
        parts.append(rowbase + fi * 8)
        x = jnp.where(chunkio == fi, MASKV, x)
    parts.append(jnp.zeros((16 - K, RB), jnp.int32))
    idx_ref[...] = jnp.concatenate(parts, axis=0)


def _phase2(cmax):
    grid = B // RB
    return pl.pallas_call(
        _p2_body,
        grid=(grid,),
        in_specs=[pl.BlockSpec((G, RB), lambda i: (0, i))],
        out_specs=pl.BlockSpec((16, RB), lambda i: (0, i)),
        out_shape=jax.ShapeDtypeStruct((16, B), jnp.int32),
        compiler_params=pltpu.CompilerParams(
            dimension_semantics=("arbitrary",),
        ),
    )(cmax)


def _sc_gather(table, flat_idx):
    n_idx = B * K
    info = plsc.get_sparse_core_info()
    nw = info.num_cores * info.num_subcores
    per_w = n_idx // nw
    n_grp = per_w // CHUNK
    mesh = plsc.VectorSubcoreMesh(core_axis_name="c", subcore_axis_name="s")

    @functools.partial(
        pl.kernel,
        mesh=mesh,
        out_type=jax.ShapeDtypeStruct((n_idx, CHUNK), jnp.int32),
        scratch_types=[
            pltpu.VMEM((CHUNK,), jnp.int32),
            pltpu.VMEM((CHUNK,), jnp.int32),
            pltpu.VMEM((CHUNK, CHUNK), jnp.int32),
            pltpu.VMEM((CHUNK, CHUNK), jnp.int32),
            pltpu.SemaphoreType.DMA,
            pltpu.SemaphoreType.DMA,
        ],
    )
    def gk(table_hbm, idx_hbm, out_hbm, idx0, idx1, rows0, rows1, sem0, sem1):
        wid = jax.lax.axis_index("s") * info.num_cores + jax.lax.axis_index("c")
        base = wid * per_w
        idxs, rows, sems = (idx0, idx1), (rows0, rows1), (sem0, sem1)
        pltpu.sync_copy(idx_hbm.at[pl.ds(base, CHUNK)], idx0)
        prev = pltpu.async_copy(table_hbm.at[idx0], rows0, sem0)
        for j in range(1, n_grp):
            b = j % 2
            off = base + j * CHUNK
            pltpu.sync_copy(idx_hbm.at[pl.ds(off, CHUNK)], idxs[b])
            cur = pltpu.async_copy(table_hbm.at[idxs[b]], rows[b], sems[b])
            prev.wait()
            pltpu.sync_copy(rows[1 - b], out_hbm.at[pl.ds(off - CHUNK, CHUNK)])
            prev = cur
        prev.wait()
        pltpu.sync_copy(rows[(n_grp - 1) % 2],
                        out_hbm.at[pl.ds(base + (n_grp - 1) * CHUNK, CHUNK)])

    return gk(table, flat_idx)


def _p4_body(cand_ref, pd_ref, out_ref):
    i = pl.program_id(0)
    cw = cand_ref[...]
    par = jax.lax.broadcasted_iota(jnp.int32, (RB, 1), 0) % 2
    bits = jnp.where(par == 0, cw << 16, cw & jnp.int32(-65536))
    x = jax.lax.bitcast_convert_type(bits, jnp.float32) * TEMP_INV
    pos = jnp.sum(pd_ref[...], axis=1, keepdims=True) * TEMP_INV
    colio = jax.lax.broadcasted_iota(jnp.int32, (RB, K * CHUNK), 1)
    vals = []
    for k in range(K):
        m = jnp.max(x, axis=1, keepdims=True)
        vals.append(m)
        if k < K - 1:
            fi = jnp.min(jnp.where(x == m, colio, BIGI), axis=1, keepdims=True)
            x = jnp.where(colio == fi, MASKV, x)
    mx = jnp.maximum(vals[0], pos)
    ssum = jnp.exp(pos - mx)
    for k in range(K):
        ssum = ssum + jnp.exp(vals[k] - mx)
    per_row = mx + jnp.log(ssum) - pos
    contrib = (jnp.sum(per_row) * (1.0 / B)).reshape(1, 1)

    @pl.when(i == 0)
    def _():
        out_ref[...] = jnp.zeros((1, 1), jnp.float32)

    out_ref[...] += contrib


def _phase4(cand, posdot):
    grid = B // RB
    return pl.pallas_call(
        _p4_body,
        grid=(grid,),
        in_specs=[
            pl.BlockSpec((RB, K * CHUNK), lambda i: (i, 0)),
            pl.BlockSpec((RB, D), lambda i: (i, 0)),
        ],
        out_specs=pl.BlockSpec((1, 1), lambda i: (0, 0)),
        out_shape=jax.ShapeDtypeStruct((1, 1), jnp.float32),
        compiler_params=pltpu.CompilerParams(
            dimension_semantics=("arbitrary",),
        ),
    )(cand, posdot)


def kernel(anchor, positive, negative_pool):
    n_pad = jnp.pad(negative_pool, ((0, N_PAD - N), (0, 0)))
    sims, cmax, posdot = _phase1(anchor, positive, n_pad)
    idx16 = _phase2(cmax)
    flat_idx = idx16[:K, :].T.reshape(-1)
    table = sims.reshape(-1, CHUNK)
    cand = _sc_gather(table, flat_idx)
    loss = _phase4(cand.reshape(B, K * CHUNK), posdot)
    return loss[0, 0]


# speedup vs baseline: 1.0425x; 1.0133x over previous
import functools

import jax
import jax.numpy as jnp
from jax.experimental import pallas as pl
from jax.experimental.pallas import tpu as pltpu
from jax.experimental.pallas import tpu_sc as plsc

TEMP_INV = 1.0 / 0.07
B = 4096
D = 128
N = 100000
CHUNK = 128
N_PAD = 100352
G = N_PAD // CHUNK
CB = 2048
GC = CB // CHUNK
K = 10
RB = 512
HB = B
NEG = -1.0e30
MASKV = -3.0e38
BIGI = 1 << 30


def _l2n(x):
    return x * (1.0 / jnp.maximum(jnp.sqrt(jnp.sum(x * x, axis=1, keepdims=True)), 1e-12))


def _p1_body(a_ref, p_ref, n_ref, sims_ref, cmax_ref, pd_ref, an_scr):
    j = pl.program_id(0)

    @pl.when(j == 0)
    def _():
        an = _l2n(a_ref[...])
        pd_ref[...] = an * _l2n(p_ref[...])
        an_scr[...] = an.astype(jnp.bfloat16)

    n = n_ref[...]
    nn = _l2n(n).astype(jnp.bfloat16)
    col = j * CB + jax.lax.broadcasted_iota(jnp.int32, (1, CB), 1)
    an = an_scr[...]
    for h in range(B // HB):
        s = jax.lax.dot_general(
            an[h * HB:(h + 1) * HB, :], nn, (((1,), (1,)), ((), ())),
            preferred_element_type=jnp.float32,
        ).astype(jnp.bfloat16)
        s = jnp.where(col < N, s, jnp.bfloat16(NEG))
        maxcols = []
        hb16 = HB // 16
        for c in range(GC):
            sc_vals = s[:, c * CHUNK:(c + 1) * CHUNK]
            w = pltpu.bitcast(sc_vals, jnp.int32)
            sims_ref[h * hb16:(h + 1) * hb16, c, :, :] = w.reshape(hb16, 8, CHUNK)
            maxcols.append(jnp.max(sc_vals, axis=1, keepdims=True).astype(jnp.float32))
        cmax_ref[:, h * HB:(h + 1) * HB] = jnp.transpose(
            jnp.concatenate(maxcols, axis=1))


def _phase1(anchor, positive, n_pad):
    grid = N_PAD // CB
    return pl.pallas_call(
        _p1_body,
        grid=(grid,),
        in_specs=[
            pl.BlockSpec((B, D), lambda j: (0, 0)),
            pl.BlockSpec((B, D), lambda j: (0, 0)),
            pl.BlockSpec((CB, D), lambda j: (j, 0)),
        ],
        out_specs=(
            pl.BlockSpec((B // 16, GC, 8, CHUNK), lambda j: (0, j, 0, 0)),
            pl.BlockSpec((GC, B), lambda j: (j, 0)),
            pl.BlockSpec((B, D), lambda j: (0, 0)),
        ),
        out_shape=(
            jax.ShapeDtypeStruct((B // 16, G, 8, CHUNK), jnp.int32),
            jax.ShapeDtypeStruct((G, B), jnp.float32),
            jax.ShapeDtypeStruct((B, D), jnp.float32),
        ),
        scratch_shapes=[pltpu.VMEM((B, D), jnp.bfloat16)],
        compiler_params=pltpu.CompilerParams(
            dimension_semantics=("arbitrary",),
        ),
    )(anchor, positive, n_pad)


def _p2_body(cmax_ref, idx_ref):
    i = pl.program_id(0)
    x = cmax_ref[...]
    chunkio = jax.lax.broadcasted_iota(jnp.int32, (G, RB), 0)
    rows = i * RB + jax.lax.broadcasted_iota(jnp.int32, (1, RB), 1)
    rowbase = (rows // 16) * (G * 8) + (rows % 16) // 2
    parts = []
    for _ in range(K):
        m = jnp.max(x, axis=0, keepdims=True)
        fi = jnp.min(jnp.where(x == m, chunkio, BIGI), axis=0, keepdims=True)
        parts.append(rowbase + fi * 8)
        x = jnp.where(chunkio == fi, MASKV, x)
    parts.append(jnp.zeros((16 - K, RB), jnp.int32))
    idx_ref[...] = jnp.concatenate(parts, axis=0)


def _phase2(cmax):
    grid = B // RB
    return pl.pallas_call(
        _p2_body,
        grid=(grid,),
        in_specs=[pl.BlockSpec((G, RB), lambda i: (0, i))],
        out_specs=pl.BlockSpec((16, RB), lambda i: (0, i)),
        out_shape=jax.ShapeDtypeStruct((16, B), jnp.int32),
        compiler_params=pltpu.CompilerParams(
            dimension_semantics=("arbitrary",),
        ),
    )(cmax)


def _sc_gather(table, flat_idx):
    n_idx = B * K
    info = plsc.get_sparse_core_info()
    nw = info.num_cores * info.num_subcores
    per_w = n_idx // nw
    n_grp = per_w // CHUNK
    mesh = plsc.VectorSubcoreMesh(core_axis_name="c", subcore_axis_name="s")

    @functools.partial(
        pl.kernel,
        mesh=mesh,
        out_type=jax.ShapeDtypeStruct((n_idx, CHUNK), jnp.int32),
        scratch_types=[
            pltpu.VMEM((CHUNK,), jnp.int32),
            pltpu.VMEM((CHUNK,), jnp.int32),
            pltpu.VMEM((CHUNK, CHUNK), jnp.int32),
            pltpu.VMEM((CHUNK, CHUNK), jnp.int32),
            pltpu.SemaphoreType.DMA,
            pltpu.SemaphoreType.DMA,
        ],
    )
    def gk(table_hbm, idx_hbm, out_hbm, idx0, idx1, rows0, rows1, sem0, sem1):
        wid = jax.lax.axis_index("s") * info.num_cores + jax.lax.axis_index("c")
        base = wid * per_w
        idxs, rows, sems = (idx0, idx1), (rows0, rows1), (sem0, sem1)
        pltpu.sync_copy(idx_hbm.at[pl.ds(base, CHUNK)], idx0)
        prev = pltpu.async_copy(table_hbm.at[idx0], rows0, sem0)
        for j in range(1, n_grp):
            b = j % 2
            off = base + j * CHUNK
            pltpu.sync_copy(idx_hbm.at[pl.ds(off, CHUNK)], idxs[b])
            cur = pltpu.async_copy(table_hbm.at[idxs[b]], rows[b], sems[b])
            prev.wait()
            pltpu.sync_copy(rows[1 - b], out_hbm.at[pl.ds(off - CHUNK, CHUNK)])
            prev = cur
        prev.wait()
        pltpu.sync_copy(rows[(n_grp - 1) % 2],
                        out_hbm.at[pl.ds(base + (n_grp - 1) * CHUNK, CHUNK)])

    return gk(table, flat_idx)


def _p4_body(cand_ref, pd_ref, out_ref):
    i = pl.program_id(0)
    cw = cand_ref[...]
    par = jax.lax.broadcasted_iota(jnp.int32, (RB, 1), 0) % 2
    bits = jnp.where(par == 0, cw << 16, cw & jnp.int32(-65536))
    x = jax.lax.bitcast_convert_type(bits, jnp.float32) * TEMP_INV
    pos = jnp.sum(pd_ref[...], axis=1, keepdims=True) * TEMP_INV
    colio = jax.lax.broadcasted_iota(jnp.int32, (RB, K * CHUNK), 1)
    vals = []
    for k in range(K):
        m = jnp.max(x, axis=1, keepdims=True)
        vals.append(m)
        if k < K - 1:
            fi = jnp.min(jnp.where(x == m, colio, BIGI), axis=1, keepdims=True)
            x = jnp.where(colio == fi, MASKV, x)
    mx = jnp.maximum(vals[0], pos)
    ssum = jnp.exp(pos - mx)
    for k in range(K):
        ssum = ssum + jnp.exp(vals[k] - mx)
    per_row = mx + jnp.log(ssum) - pos
    contrib = (jnp.sum(per_row) * (1.0 / B)).reshape(1, 1)

    @pl.when(i == 0)
    def _():
        out_ref[...] = jnp.zeros((1, 1), jnp.float32)

    out_ref[...] += contrib


def _phase4(cand, posdot):
    grid = B // RB
    return pl.pallas_call(
        _p4_body,
        grid=(grid,),
        in_specs=[
            pl.BlockSpec((RB, K * CHUNK), lambda i: (i, 0)),
            pl.BlockSpec((RB, D), lambda i: (i, 0)),
        ],
        out_specs=pl.BlockSpec((1, 1), lambda i: (0, 0)),
        out_shape=jax.ShapeDtypeStruct((1, 1), jnp.float32),
        compiler_params=pltpu.CompilerParams(
            dimension_semantics=("arbitrary",),
        ),
    )(cand, posdot)


def kernel(anchor, positive, negative_pool):
    n_pad = jnp.pad(negative_pool, ((0, N_PAD - N), (0, 0)))
    sims, cmax, posdot = _phase1(anchor, positive, n_pad)
    idx16 = _phase2(cmax)
    flat_idx = idx16[:K, :].T.reshape(-1)
    table = sims.reshape(-1, CHUNK)
    cand = _sc_gather(table, flat_idx)
    loss = _phase4(cand.reshape(B, K * CHUNK), posdot)
    return loss[0, 0]
